# baseline (speedup 1.0000x reference)
import math

import jax
import jax.numpy as jnp
from jax import lax
from jax.experimental import pallas as pl
from jax.experimental.pallas import tpu as pltpu

_RC = 5.0
_K = 16
_CI = 32


def _logits_body(h_ref, cf_ref, Wa_ref, out_ref):
    ma = h_ref.shape[1]
    nfs = h_ref.shape[2]
    hb = h_ref[0]
    cfb = cf_ref[0]
    for i0 in range(0, ma, _CI):
        hcb = hb[i0:i0 + _CI, :]
        cfc = cfb[i0:i0 + _CI, :]
        u1 = jnp.broadcast_to(hcb[:, None, :], (_CI, ma, nfs))
        u2 = hb[None, :, :] * cfc[:, :, None]
        u = jnp.concatenate([u1, u2], axis=2)
        gch = lax.dot_general(u, Wa_ref[:, :], (((2,), (0,)), ((), ())))
        out_ref[0, i0:i0 + _CI, :] = gch[:, :, 0]


def _pool_body(att_ref, d_ref, r_ref, rT_ref, h_ref, Wp_ref, bp_ref,
               out_ref):
    ma = att_ref.shape[1]
    nfs = h_ref.shape[2]
    nout = Wp_ref.shape[1]

    att = att_ref[0]
    d = d_ref[0]
    rb = r_ref[0]
    rTb = rT_ref[0]
    hb = h_ref[0]
    bp = bp_ref[0:1, :]

    rx_c = rb[:, 0:1]
    ry_c = rb[:, 1:2]
    rz_c = rb[:, 2:3]
    rx_r = rTb[0:1, :]
    ry_r = rTb[1:2, :]
    rz_r = rTb[2:3, :]

    jj = lax.broadcasted_iota(jnp.int32, (ma, ma), 1)

    rr = lax.broadcasted_iota(jnp.int32, (3 * nout, 3 * nout), 0)
    cc = lax.broadcasted_iota(jnp.int32, (3 * nout, 3 * nout), 1)
    M = (cc == 3 * (rr % nout) + rr // nout).astype(jnp.float32)

    P1 = jnp.dot(hb, Wp_ref[0:nfs, :])
    P2 = jnp.dot(hb, Wp_ref[nfs:2 * nfs, :])

    work = att
    for kk in range(_K):
        m = jnp.max(work, axis=1, keepdims=True)
        ismax = work == m
        idxsel = jnp.min(jnp.where(ismax, jj, ma), axis=1, keepdims=True)
        sel = jj == idxsel
        selF = sel.astype(jnp.float32)

        d_k = jnp.sum(jnp.where(sel, d, 0.0), axis=1, keepdims=True)
        rxg = jnp.sum(selF * rx_r, axis=1, keepdims=True)
        ryg = jnp.sum(selF * ry_r, axis=1, keepdims=True)
        rzg = jnp.sum(selF * rz_r, axis=1, keepdims=True)
        dclip = jnp.maximum(d_k, 1e-4)
        ex = (rx_c - rxg) / dclip
        ey = (ry_c - ryg) / dclip
        ez = (rz_c - rzg) / dclip

        p2g = jnp.dot(selF, P2)
        q = m * (P1 + p2g) + bp
        G = jnp.concatenate([q * ex, q * ey, q * ez], axis=1)
        out_ref[0, :, kk * 3 * nout:(kk + 1) * 3 * nout] = jnp.dot(G, M)

        work = jnp.where(sel, -1.0, work)


def kernel(z, r, h, W_att, b_att, W_pool, b_pool):
    n, ma = z.shape
    nfs = h.shape[2]
    nout = W_pool.shape[1]

    dv = r[:, :, None, :] - r[:, None, :, :]
    d = jnp.sqrt(jnp.sum(dv * dv, axis=3) + 1e-12)
    cutoff = 0.5 * (jnp.cos(math.pi * jnp.minimum(d, _RC) / _RC) + 1.0)
    eye = jnp.eye(ma, dtype=jnp.float32)
    cf = (cutoff - eye[None, :, :])[..., None]
    mask = (z > -1).astype(jnp.float32)
    mask = mask[:, None, :] * mask[:, :, None]
    mask = jnp.clip(mask - eye[None, :, :], 0.0, None)[..., None]
    cf = cf * mask

    g = pl.pallas_call(
        _logits_body,
        grid=(n,),
        in_specs=[
            pl.BlockSpec((1, ma, nfs), lambda b: (b, 0, 0)),
            pl.BlockSpec((1, ma, ma), lambda b: (b, 0, 0)),
            pl.BlockSpec((2 * nfs, 1), lambda b: (0, 0)),
        ],
        out_specs=pl.BlockSpec((1, ma, ma), lambda b: (b, 0, 0)),
        out_shape=jax.ShapeDtypeStruct((n, ma, ma), jnp.float32),
    )(h, cf[..., 0], W_att)

    gm = (g[..., None] + b_att) * mask
    gm = jnp.exp(gm) - (1.0 - mask)
    attention = gm / jnp.clip(jnp.sum(gm, axis=2, keepdims=True), 1e-8, None)

    rT = jnp.swapaxes(r, 1, 2)
    bpool = b_pool.reshape(1, nout)
    out = pl.pallas_call(
        _pool_body,
        grid=(n,),
        in_specs=[
            pl.BlockSpec((1, ma, ma), lambda b: (b, 0, 0)),
            pl.BlockSpec((1, ma, ma), lambda b: (b, 0, 0)),
            pl.BlockSpec((1, ma, 3), lambda b: (b, 0, 0)),
            pl.BlockSpec((1, 3, ma), lambda b: (b, 0, 0)),
            pl.BlockSpec((1, ma, nfs), lambda b: (b, 0, 0)),
            pl.BlockSpec((2 * nfs, nout), lambda b: (0, 0)),
            pl.BlockSpec((1, nout), lambda b: (0, 0)),
        ],
        out_specs=pl.BlockSpec((1, ma, _K * 3 * nout), lambda b: (b, 0, 0)),
        out_shape=jax.ShapeDtypeStruct((n, ma, _K * 3 * nout), jnp.float32),
    )(attention[..., 0], d, r, rT, h, W_pool, bpool)

    c = out.reshape(n, ma, _K, nout, 3)
    return (z, r, c)


# speedup vs baseline: 1.2511x; 1.2511x over previous
import math

import jax
import jax.numpy as jnp
from jax import lax
from jax.experimental import pallas as pl
from jax.experimental.pallas import tpu as pltpu

_RC = 5.0
_K = 16
_CI = 32


def _logits_body(h_ref, cf_ref, Wa_ref, out_ref):
    ma = h_ref.shape[1]
    nfs = h_ref.shape[2]
    hb = h_ref[0]
    cfb = cf_ref[0]
    for i0 in range(0, ma, _CI):
        hcb = hb[i0:i0 + _CI, :]
        cfc = cfb[i0:i0 + _CI, :]
        u1 = jnp.broadcast_to(hcb[:, None, :], (_CI, ma, nfs))
        u2 = hb[None, :, :] * cfc[:, :, None]
        u = jnp.concatenate([u1, u2], axis=2)
        gch = lax.dot_general(u, Wa_ref[:, :], (((2,), (0,)), ((), ())))
        out_ref[0, i0:i0 + _CI, :] = gch[:, :, 0]


def _pool_body(att_ref, d_ref, r_ref, rT_ref, h_ref, Wp_ref, bp_ref,
               q_ref, e_ref):
    ma = att_ref.shape[1]
    nfs = h_ref.shape[2]
    nout = Wp_ref.shape[1]

    att = att_ref[0]
    d = d_ref[0]
    rb = r_ref[0]
    rTb = rT_ref[0]
    hb = h_ref[0]
    bp = bp_ref[0:1, :]

    rx_c = rb[:, 0:1]
    ry_c = rb[:, 1:2]
    rz_c = rb[:, 2:3]
    rx_r = rTb[0:1, :]
    ry_r = rTb[1:2, :]
    rz_r = rTb[2:3, :]

    jj = lax.broadcasted_iota(jnp.int32, (ma, ma), 1)

    P1 = jnp.dot(hb, Wp_ref[0:nfs, :])
    P2 = jnp.dot(hb, Wp_ref[nfs:2 * nfs, :])

    work = att
    for kk in range(_K):
        m = jnp.max(work, axis=1, keepdims=True)
        ismax = work == m
        idxsel = jnp.min(jnp.where(ismax, jj, ma), axis=1, keepdims=True)
        sel = jj == idxsel
        selF = sel.astype(jnp.float32)

        d_k = jnp.sum(jnp.where(sel, d, 0.0), axis=1, keepdims=True)
        rxg = jnp.sum(selF * rx_r, axis=1, keepdims=True)
        ryg = jnp.sum(selF * ry_r, axis=1, keepdims=True)
        rzg = jnp.sum(selF * rz_r, axis=1, keepdims=True)
        dclip = jnp.maximum(d_k, 1e-4)
        ex = (rx_c - rxg) / dclip
        ey = (ry_c - ryg) / dclip
        ez = (rz_c - rzg) / dclip

        p2g = jnp.dot(selF, P2)
        q_ref[0, kk] = m * (P1 + p2g) + bp
        e_ref[0, 0, :, kk:kk + 1] = ex
        e_ref[0, 1, :, kk:kk + 1] = ey
        e_ref[0, 2, :, kk:kk + 1] = ez

        work = jnp.where(sel, -1.0, work)


def kernel(z, r, h, W_att, b_att, W_pool, b_pool):
    n, ma = z.shape
    nfs = h.shape[2]
    nout = W_pool.shape[1]

    dv = r[:, :, None, :] - r[:, None, :, :]
    d = jnp.sqrt(jnp.sum(dv * dv, axis=3) + 1e-12)
    cutoff = 0.5 * (jnp.cos(math.pi * jnp.minimum(d, _RC) / _RC) + 1.0)
    eye = jnp.eye(ma, dtype=jnp.float32)
    cf = (cutoff - eye[None, :, :])[..., None]
    mask = (z > -1).astype(jnp.float32)
    mask = mask[:, None, :] * mask[:, :, None]
    mask = jnp.clip(mask - eye[None, :, :], 0.0, None)[..., None]
    cf = cf * mask

    g = pl.pallas_call(
        _logits_body,
        grid=(n,),
        in_specs=[
            pl.BlockSpec((1, ma, nfs), lambda b: (b, 0, 0)),
            pl.BlockSpec((1, ma, ma), lambda b: (b, 0, 0)),
            pl.BlockSpec((2 * nfs, 1), lambda b: (0, 0)),
        ],
        out_specs=pl.BlockSpec((1, ma, ma), lambda b: (b, 0, 0)),
        out_shape=jax.ShapeDtypeStruct((n, ma, ma), jnp.float32),
    )(h, cf[..., 0], W_att)

    gm = (g[..., None] + b_att) * mask
    gm = jnp.exp(gm) - (1.0 - mask)
    attention = gm / jnp.clip(jnp.sum(gm, axis=2, keepdims=True), 1e-8, None)

    rT = jnp.swapaxes(r, 1, 2)
    bpool = b_pool.reshape(1, nout)
    out = pl.pallas_call(
        _pool_body,
        grid=(n,),
        in_specs=[
            pl.BlockSpec((1, ma, ma), lambda b: (b, 0, 0)),
            pl.BlockSpec((1, ma, ma), lambda b: (b, 0, 0)),
            pl.BlockSpec((1, ma, 3), lambda b: (b, 0, 0)),
            pl.BlockSpec((1, 3, ma), lambda b: (b, 0, 0)),
            pl.BlockSpec((1, ma, nfs), lambda b: (b, 0, 0)),
            pl.BlockSpec((2 * nfs, nout), lambda b: (0, 0)),
            pl.BlockSpec((1, nout), lambda b: (0, 0)),
        ],
        out_specs=[
            pl.BlockSpec((1, _K, ma, nout), lambda b: (b, 0, 0, 0)),
            pl.BlockSpec((1, 3, ma, _K), lambda b: (b, 0, 0, 0)),
        ],
        out_shape=[
            jax.ShapeDtypeStruct((n, _K, ma, nout), jnp.float32),
            jax.ShapeDtypeStruct((n, 3, ma, _K), jnp.float32),
        ],
    )(attention[..., 0], d, r, rT, h, W_pool, bpool)
    q_out, e_out = out

    q4 = jnp.transpose(q_out, (0, 2, 1, 3))
    e4 = jnp.transpose(e_out, (0, 2, 3, 1))
    c = q4[..., None] * e4[:, :, :, None, :]
    return (z, r, c)


# speedup vs baseline: 1.2523x; 1.0010x over previous
import math

import jax
import jax.numpy as jnp
from jax import lax
from jax.experimental import pallas as pl
from jax.experimental.pallas import tpu as pltpu

_RC = 5.0
_K = 16
_CI = 32


def _logits_body(h_ref, cf_ref, Wa_ref, out_ref):
    ma = h_ref.shape[1]
    nfs = h_ref.shape[2]
    hb = h_ref[0]
    cfb = cf_ref[0]
    for i0 in range(0, ma, _CI):
        hcb = hb[i0:i0 + _CI, :]
        cfc = cfb[i0:i0 + _CI, :]
        u1 = jnp.broadcast_to(hcb[:, None, :], (_CI, ma, nfs))
        u2 = hb[None, :, :] * cfc[:, :, None]
        u = jnp.concatenate([u1, u2], axis=2)
        gch = lax.dot_general(u, Wa_ref[:, :], (((2,), (0,)), ((), ())))
        out_ref[0, i0:i0 + _CI, :] = gch[:, :, 0]


def _pool_body(att_ref, d_ref, r_ref, rT_ref, h_ref, Wp_ref, bp_ref,
               q_ref, e_ref):
    ma = att_ref.shape[1]
    nfs = h_ref.shape[2]
    nout = Wp_ref.shape[1]

    att = att_ref[0]
    d = d_ref[0]
    rb = r_ref[0]
    rTb = rT_ref[0]
    hb = h_ref[0]
    bp = bp_ref[0:1, :]

    rx_c = rb[:, 0:1]
    ry_c = rb[:, 1:2]
    rz_c = rb[:, 2:3]
    rx_r = rTb[0:1, :]
    ry_r = rTb[1:2, :]
    rz_r = rTb[2:3, :]

    jj = lax.broadcasted_iota(jnp.int32, (ma, ma), 1)

    P1 = jnp.dot(hb, Wp_ref[0:nfs, :])
    P2 = jnp.dot(hb, Wp_ref[nfs:2 * nfs, :])

    work = att
    for kk in range(_K):
        m = jnp.max(work, axis=1, keepdims=True)
        ismax = work == m
        idxsel = jnp.min(jnp.where(ismax, jj, ma), axis=1, keepdims=True)
        sel = jj == idxsel
        selF = sel.astype(jnp.float32)

        d_k = jnp.sum(jnp.where(sel, d, 0.0), axis=1, keepdims=True)
        rxg = jnp.sum(selF * rx_r, axis=1, keepdims=True)
        ryg = jnp.sum(selF * ry_r, axis=1, keepdims=True)
        rzg = jnp.sum(selF * rz_r, axis=1, keepdims=True)
        dclip = jnp.maximum(d_k, 1e-4)
        ex = (rx_c - rxg) / dclip
        ey = (ry_c - ryg) / dclip
        ez = (rz_c - rzg) / dclip

        p2g = jnp.dot(selF, P2)
        q_ref[0, :, kk * nout:(kk + 1) * nout] = m * (P1 + p2g) + bp
        e_ref[0, 0, :, kk:kk + 1] = ex
        e_ref[0, 1, :, kk:kk + 1] = ey
        e_ref[0, 2, :, kk:kk + 1] = ez

        work = jnp.where(sel, -1.0, work)


def kernel(z, r, h, W_att, b_att, W_pool, b_pool):
    n, ma = z.shape
    nfs = h.shape[2]
    nout = W_pool.shape[1]

    dv = r[:, :, None, :] - r[:, None, :, :]
    d = jnp.sqrt(jnp.sum(dv * dv, axis=3) + 1e-12)
    cutoff = 0.5 * (jnp.cos(math.pi * jnp.minimum(d, _RC) / _RC) + 1.0)
    eye = jnp.eye(ma, dtype=jnp.float32)
    cf = (cutoff - eye[None, :, :])[..., None]
    mask = (z > -1).astype(jnp.float32)
    mask = mask[:, None, :] * mask[:, :, None]
    mask = jnp.clip(mask - eye[None, :, :], 0.0, None)[..., None]
    cf = cf * mask

    g = pl.pallas_call(
        _logits_body,
        grid=(n,),
        in_specs=[
            pl.BlockSpec((1, ma, nfs), lambda b: (b, 0, 0)),
            pl.BlockSpec((1, ma, ma), lambda b: (b, 0, 0)),
            pl.BlockSpec((2 * nfs, 1), lambda b: (0, 0)),
        ],
        out_specs=pl.BlockSpec((1, ma, ma), lambda b: (b, 0, 0)),
        out_shape=jax.ShapeDtypeStruct((n, ma, ma), jnp.float32),
    )(h, cf[..., 0], W_att)

    gm = (g[..., None] + b_att) * mask
    gm = jnp.exp(gm) - (1.0 - mask)
    attention = gm / jnp.clip(jnp.sum(gm, axis=2, keepdims=True), 1e-8, None)

    rT = jnp.swapaxes(r, 1, 2)
    bpool = b_pool.reshape(1, nout)
    out = pl.pallas_call(
        _pool_body,
        grid=(n,),
        in_specs=[
            pl.BlockSpec((1, ma, ma), lambda b: (b, 0, 0)),
            pl.BlockSpec((1, ma, ma), lambda b: (b, 0, 0)),
            pl.BlockSpec((1, ma, 3), lambda b: (b, 0, 0)),
            pl.BlockSpec((1, 3, ma), lambda b: (b, 0, 0)),
            pl.BlockSpec((1, ma, nfs), lambda b: (b, 0, 0)),
            pl.BlockSpec((2 * nfs, nout), lambda b: (0, 0)),
            pl.BlockSpec((1, nout), lambda b: (0, 0)),
        ],
        out_specs=[
            pl.BlockSpec((1, ma, _K * nout), lambda b: (b, 0, 0)),
            pl.BlockSpec((1, 3, ma, _K), lambda b: (b, 0, 0, 0)),
        ],
        out_shape=[
            jax.ShapeDtypeStruct((n, ma, _K * nout), jnp.float32),
            jax.ShapeDtypeStruct((n, 3, ma, _K), jnp.float32),
        ],
    )(attention[..., 0], d, r, rT, h, W_pool, bpool)
    q_out, e_out = out

    q4 = q_out.reshape(n, ma, _K, nout)
    e4 = jnp.transpose(e_out, (0, 2, 3, 1))
    c = q4[..., None] * e4[:, :, :, None, :]
    return (z, r, c)
